# Initial kernel scaffold; baseline (speedup 1.0000x reference)
import jax
import jax.numpy as jnp
from jax.experimental import pallas as pl


def kernel(x, y, edge_index, edge_attr, batch, emb, We1, be1, W1a, b1a, g1, bt1, W1b, b1b, We2, be2, W2a, b2a, g2, bt2, W2b, b2b, Wf1, bf1, Wf2, bf2, Wb1, bb1, Wb2, bb2):
    raise NotImplementedError("write your pallas kernel here")


# speedup vs baseline: 3.6155x; 3.6155x over previous
import functools

import jax
import jax.numpy as jnp
from jax import lax
from jax.experimental import pallas as pl
from jax.experimental.pallas import tpu as pltpu
from jax.experimental.pallas import tpu_sc as plsc

N = 10000
E = 160000
G = 128
NT = 512
DH = 256

NC = 2
NS = 16
NW = NC * NS

_mesh = functools.partial(
    plsc.VectorSubcoreMesh,
    core_axis_name="c",
    subcore_axis_name="s",
    num_cores=NC,
    num_subcores=NS,
)

SUBQ = 632


def _splat16(val):
    return jnp.zeros((16,), jnp.int32) + val


def _lane(vec16, j):
    return lax.gather(
        vec16,
        _splat16(j)[:, None],
        lax.GatherDimensionNumbers(
            offset_dims=(), collapsed_slice_dims=(0,), start_index_map=(0,)),
        slice_sizes=(1,),
        mode=lax.GatherScatterMode.PROMISE_IN_BOUNDS,
    )


def _emb_gather(embp, xidx, y):
    QUOTA = 320
    CH = 80

    @functools.partial(
        pl.kernel,
        out_type=jax.ShapeDtypeStruct((N, 128), jnp.float32),
        mesh=_mesh(),
        scratch_types=[
            pltpu.VMEM((CH,), jnp.int32),
            pltpu.VMEM((CH,), jnp.float32),
            pltpu.VMEM((CH, 128), jnp.float32),
            pltpu.SemaphoreType.DMA,
        ],
    )
    def k(emb_h, idx_h, y_h, out_h, idxv, yv, rows, sem):
        cid = lax.axis_index("c")
        sid = lax.axis_index("s")
        wid = sid * NC + cid
        base = jnp.minimum(wid * QUOTA, N - QUOTA)
        for kk in range(QUOTA // CH):
            b2 = base + kk * CH
            pltpu.sync_copy(idx_h.at[pl.ds(b2, CH)], idxv)
            pltpu.sync_copy(y_h.at[pl.ds(b2, CH)], yv)
            pltpu.async_copy(emb_h.at[idxv], rows, sem).wait()
            lastlane = lax.iota(jnp.int32, 16) == 15
            sl = pl.ds(112, 16)
            for g in range(CH // 16):
                y16 = yv[pl.ds(g * 16, 16)]
                for j in range(16):
                    r = g * 16 + j
                    rows[r, sl] = jnp.where(lastlane, _lane(y16, j), rows[r, sl])
            pltpu.sync_copy(rows, out_h.at[pl.ds(b2, CH)])

    return k(embp, xidx, y)


def _edge_compute(rows, eav, wv, bv):
    def grp(g, carry):
        a16 = eav[pl.ds(g * 16, 16)]
        for j in range(16):
            e = g * 16 + j
            aj = _lane(a16, j)
            for c2 in range(8):
                sl = pl.ds(16 * c2, 16)
                r = rows[e, sl]
                rows[e, sl] = jnp.maximum(r + aj * wv[c2] + bv[c2], 0.0)
        return carry

    lax.fori_loop(0, 8, grp, 0)


def _edge_pass1(xy, src, dst, ea, wb, zeros):
    NCHUNK = E // 128

    @functools.partial(
        pl.kernel,
        out_type=jax.ShapeDtypeStruct((2, N, 128), jnp.float32),
        mesh=_mesh(),
        scratch_types=[
            pltpu.VMEM_SHARED((N, 128), jnp.float32),
            pltpu.VMEM((128,), jnp.int32),
            pltpu.VMEM((128,), jnp.int32),
            pltpu.VMEM((128,), jnp.float32),
            pltpu.VMEM((128, 128), jnp.float32),
            pltpu.VMEM((2, 128), jnp.float32),
            pltpu.SemaphoreType.DMA,
        ],
    )
    def k(xy_h, src_h, dst_h, ea_h, wb_h, z_h, out_h,
          aggr, srcv, dstv, eav, rows, wbv, sem):
        cid = lax.axis_index("c")
        sid = lax.axis_index("s")
        wid = sid * NC + cid
        r0 = jnp.minimum(sid * SUBQ, N - SUBQ)
        pltpu.sync_copy(z_h.at[pl.ds(r0, SUBQ)], aggr.at[pl.ds(r0, SUBQ)])
        pltpu.sync_copy(wb_h, wbv)
        plsc.subcore_barrier()
        wv = [wbv[0, pl.ds(16 * c2, 16)] for c2 in range(8)]
        bv = [wbv[1, pl.ds(16 * c2, 16)] for c2 in range(8)]
        nk = jnp.where(wid < NCHUNK % NW, NCHUNK // NW + 1, NCHUNK // NW)

        def chunk(kk, carry):
            base = (wid + kk * NW) * 128
            pltpu.sync_copy(src_h.at[pl.ds(base, 128)], srcv)
            pltpu.sync_copy(dst_h.at[pl.ds(base, 128)], dstv)
            pltpu.sync_copy(ea_h.at[pl.ds(base, 128)], eav)
            pltpu.async_copy(xy_h.at[srcv], rows, sem).wait()
            _edge_compute(rows, eav, wv, bv)
            pltpu.sync_copy(rows, aggr.at[dstv], add=True)
            return carry

        lax.fori_loop(0, nk, chunk, 0)
        plsc.subcore_barrier()
        pltpu.sync_copy(aggr.at[pl.ds(r0, SUBQ)],
                        out_h.at[cid, pl.ds(r0, SUBQ)])

    return k(xy, src, dst, ea, wb, zeros)


def _edge_pass2(h1a, h1b, src, dst, ea, wb2, zeros):
    NCHUNK = E // 128

    @functools.partial(
        pl.kernel,
        out_type=jax.ShapeDtypeStruct((2, N, 128), jnp.float32),
        mesh=_mesh(),
        scratch_types=[
            pltpu.VMEM_SHARED((N, 128), jnp.float32),
            pltpu.VMEM((128,), jnp.int32),
            pltpu.VMEM((128,), jnp.int32),
            pltpu.VMEM((128,), jnp.float32),
            pltpu.VMEM((128, 128), jnp.float32),
            pltpu.VMEM((2, 128), jnp.float32),
            pltpu.SemaphoreType.DMA,
        ],
    )
    def k(ha_h, hb_h, src_h, dst_h, ea_h, wb_h, z_h, out_h,
          aggr, srcv, dstv, eav, rows, wbv, sem):
        cid = lax.axis_index("c")
        sid = lax.axis_index("s")
        r0 = jnp.minimum(sid * SUBQ, N - SUBQ)
        pltpu.sync_copy(z_h.at[pl.ds(r0, SUBQ)], aggr.at[pl.ds(r0, SUBQ)])
        pltpu.sync_copy(wb_h.at[cid], wbv)
        plsc.subcore_barrier()
        wv = [wbv[0, pl.ds(16 * c2, 16)] for c2 in range(8)]
        bv = [wbv[1, pl.ds(16 * c2, 16)] for c2 in range(8)]
        nk = jnp.where(sid < NCHUNK % NS, NCHUNK // NS + 1, NCHUNK // NS)

        def chunk(kk, carry):
            base = (sid + kk * NS) * 128
            pltpu.sync_copy(src_h.at[pl.ds(base, 128)], srcv)
            pltpu.sync_copy(dst_h.at[pl.ds(base, 128)], dstv)
            pltpu.sync_copy(ea_h.at[pl.ds(base, 128)], eav)

            @pl.when(cid == 0)
            def _():
                pltpu.async_copy(ha_h.at[srcv], rows, sem).wait()

            @pl.when(cid == 1)
            def _():
                pltpu.async_copy(hb_h.at[srcv], rows, sem).wait()

            _edge_compute(rows, eav, wv, bv)
            pltpu.sync_copy(rows, aggr.at[dstv], add=True)
            return carry

        lax.fori_loop(0, nk, chunk, 0)
        plsc.subcore_barrier()
        pltpu.sync_copy(aggr.at[pl.ds(r0, SUBQ)],
                        out_h.at[cid, pl.ds(r0, SUBQ)])

    return k(h1a, h1b, src, dst, ea, wb2, zeros)


R = 1000
NB = N // R


def _mlp_a(parts, agg, WT, b):
    DI = WT.shape[0]

    def body(*refs):
        nparts = len(parts)
        part_refs = refs[:nparts]
        agg_r, w_r, b_r, u_r, st_r, acc_r = refs[nparts:]
        i = pl.program_id(0)
        if nparts == 1:
            z = part_refs[0][...]
        else:
            z = jnp.concatenate([p[...] for p in part_refs], axis=1)
        z = z + jnp.concatenate([agg_r[0], agg_r[1]], axis=1) \
            if DI == 256 else z + agg_r[0] + agg_r[1]
        u = jnp.dot(z, w_r[...], preferred_element_type=jnp.float32) + b_r[...]
        u_r[...] = u
        s1 = jnp.sum(u, axis=0, keepdims=True)
        s2 = jnp.sum(u * u, axis=0, keepdims=True)
        st = jnp.concatenate([s1, s2], axis=0)

        @pl.when(i == 0)
        def _():
            acc_r[...] = st

        @pl.when(i > 0)
        def _():
            acc_r[...] = acc_r[...] + st

        @pl.when(i == NB - 1)
        def _():
            st_r[...] = acc_r[...]

    in_specs = (
        [pl.BlockSpec((R, p.shape[1]), lambda i: (i, 0)) for p in parts]
        + [
            pl.BlockSpec((2, R, 128), lambda i: (0, i, 0)),
            pl.BlockSpec((DI, DH), lambda i: (0, 0)),
            pl.BlockSpec((1, DH), lambda i: (0, 0)),
        ]
    )
    return pl.pallas_call(
        body,
        grid=(NB,),
        in_specs=in_specs,
        out_specs=[
            pl.BlockSpec((R, DH), lambda i: (i, 0)),
            pl.BlockSpec((2, DH), lambda i: (0, 0)),
        ],
        out_shape=[
            jax.ShapeDtypeStruct((N, DH), jnp.float32),
            jax.ShapeDtypeStruct((2, DH), jnp.float32),
        ],
        scratch_shapes=[pltpu.VMEM((2, DH), jnp.float32)],
    )(*parts, agg, WT, b)


def _mlp_b(u, stats, g, bt, WT, b2):
    def body(u_r, st_r, g_r, bt_r, w_r, b_r, ha_r, hb_r):
        st = st_r[...]
        m = st[0:1, :] / N
        v = st[1:2, :] / N - m * m
        inv = lax.rsqrt(v + 1e-5)
        t = jnp.maximum((u_r[...] - m) * inv * g_r[...] + bt_r[...], 0.0)
        h = jnp.dot(t, w_r[...], preferred_element_type=jnp.float32) + b_r[...]
        h = jnp.maximum(h, 0.0)
        ha_r[...] = h[:, :128]
        hb_r[...] = h[:, 128:]

    return pl.pallas_call(
        body,
        grid=(NB,),
        in_specs=[
            pl.BlockSpec((R, DH), lambda i: (i, 0)),
            pl.BlockSpec((2, DH), lambda i: (0, 0)),
            pl.BlockSpec((1, DH), lambda i: (0, 0)),
            pl.BlockSpec((1, DH), lambda i: (0, 0)),
            pl.BlockSpec((DH, DH), lambda i: (0, 0)),
            pl.BlockSpec((1, DH), lambda i: (0, 0)),
        ],
        out_specs=[
            pl.BlockSpec((R, 128), lambda i: (i, 0)),
            pl.BlockSpec((R, 128), lambda i: (i, 0)),
        ],
        out_shape=[
            jax.ShapeDtypeStruct((N, 128), jnp.float32),
            jax.ShapeDtypeStruct((N, 128), jnp.float32),
        ],
    )(u, stats, g, bt, WT, b2)


def _pool_heads(batch8, h1a, h1b, h2a, h2b,
                Wf1T, bf1, Wf2T, bf2, Wb1T, bb1, Wb2T, bb2):
    def body(b_r, h1a_r, h1b_r, h2a_r, h2b_r,
             wf1_r, bf1_r, wf2_r, bf2_r, wb1_r, bb1_r, wb2_r, bb2_r,
             lf_r, lb_r, pacc):
        i = pl.program_id(0)

        @pl.when(i == 0)
        def _():
            pacc[...] = jnp.zeros((G, 2 * DH), jnp.float32)

        bb = b_r[0]
        oh = (lax.broadcasted_iota(jnp.int32, (G, R), 0) == bb)
        oh = oh.astype(jnp.float32)
        for idx, hr in enumerate((h1a_r, h1b_r, h2a_r, h2b_r)):
            sl = pl.ds(128 * idx, 128)
            pacc[:, sl] = pacc[:, sl] + jnp.dot(
                oh, hr[...], preferred_element_type=jnp.float32)

        @pl.when(i == NB - 1)
        def _():
            h = pacc[...]
            tf = jnp.maximum(
                jnp.dot(h, wf1_r[...], preferred_element_type=jnp.float32)
                + bf1_r[...], 0.0)
            lf_r[...] = jnp.dot(
                tf, wf2_r[...], preferred_element_type=jnp.float32) + bf2_r[...]
            tb = jnp.maximum(
                jnp.dot(h, wb1_r[...], preferred_element_type=jnp.float32)
                + bb1_r[...], 0.0)
            lb_r[...] = jnp.dot(
                tb, wb2_r[...], preferred_element_type=jnp.float32) + bb2_r[...]

    full = lambda shape: pl.BlockSpec(shape, lambda i: tuple(0 for _ in shape))
    return pl.pallas_call(
        body,
        grid=(NB,),
        in_specs=[
            pl.BlockSpec((1, 1, R), lambda i: (i, 0, 0)),
            pl.BlockSpec((R, 128), lambda i: (i, 0)),
            pl.BlockSpec((R, 128), lambda i: (i, 0)),
            pl.BlockSpec((R, 128), lambda i: (i, 0)),
            pl.BlockSpec((R, 128), lambda i: (i, 0)),
            full((2 * DH, DH)), full((1, DH)),
            full((DH, NT)), full((1, NT)),
            full((2 * DH, DH)), full((1, DH)),
            full((DH, NT)), full((1, NT)),
        ],
        out_specs=[full((G, NT)), full((G, NT))],
        out_shape=[
            jax.ShapeDtypeStruct((G, NT), jnp.float32),
            jax.ShapeDtypeStruct((G, NT), jnp.float32),
        ],
        scratch_shapes=[pltpu.VMEM((G, 2 * DH), jnp.float32)],
    )(batch8, h1a, h1b, h2a, h2b,
      Wf1T, bf1, Wf2T, bf2, Wb1T, bb1, Wb2T, bb2)


def kernel(x, y, edge_index, edge_attr, batch, emb, We1, be1, W1a, b1a, g1,
           bt1, W1b, b1b, We2, be2, W2a, b2a, g2, bt2, W2b, b2b, Wf1, bf1,
           Wf2, bf2, Wb1, bb1, Wb2, bb2):
    f32 = jnp.float32
    embp = jnp.pad(emb.astype(f32), ((0, 0), (0, 1)))
    xidx = x.reshape(-1).astype(jnp.int32)
    src = edge_index[0].astype(jnp.int32)
    dst = edge_index[1].astype(jnp.int32)
    ea = edge_attr.reshape(-1).astype(f32)
    zeros = jnp.zeros((N, 128), f32)

    wb1_e = jnp.stack([We1[:, 0], be1])
    wb2_e = jnp.stack(
        [jnp.stack([We2[:128, 0], be2[:128]]),
         jnp.stack([We2[128:, 0], be2[128:]])])

    xy = _emb_gather(embp, xidx, y)
    pagg1 = _edge_pass1(xy, src, dst, ea, wb1_e, zeros)

    u1, st1 = _mlp_a([xy], pagg1, W1a.T, b1a.reshape(1, -1))
    h1a, h1b = _mlp_b(u1, st1, g1.reshape(1, -1), bt1.reshape(1, -1),
                      W1b.T, b1b.reshape(1, -1))

    agg2 = _edge_pass2(h1a, h1b, src, dst, ea, wb2_e, zeros)
    u2, st2 = _mlp_a([h1a, h1b], agg2, W2a.T, b2a.reshape(1, -1))
    h2a, h2b = _mlp_b(u2, st2, g2.reshape(1, -1), bt2.reshape(1, -1),
                      W2b.T, b2b.reshape(1, -1))

    batch3 = batch.reshape(NB, 1, R).astype(jnp.int32)
    lf, lb = _pool_heads(batch3, h1a, h1b, h2a, h2b,
                         Wf1.T, bf1.reshape(1, -1), Wf2.T, bf2.reshape(1, -1),
                         Wb1.T, bb1.reshape(1, -1), Wb2.T, bb2.reshape(1, -1))
    return (lf, lb)
